# baseline (speedup 1.0000x reference)
import math

import jax
import jax.numpy as jnp
from jax import lax
from jax.experimental import pallas as pl
from jax.experimental.pallas import tpu as pltpu
from jax.experimental.pallas import tpu_sc as plsc

N = 10000
NP = 10240
E = 320000
D = 128
DH = D // 2
DP = 80
B = 128
NBLKP = 2560
EP = NBLKP * B
RSUB = NP // 16
R_TC = 2048
R_TC2 = 2000
BA = 64
NBLKA = EP // BA
NB_A = NBLKA // 32
NB_B = NBLKP // 16


def _proj_body(x_ref, wcat_ref, we_ref, qk_ref, u_ref, vlo_ref, vhi_ref):
    xb = x_ref[...]
    qkv = jnp.dot(xb, wcat_ref[...], preferred_element_type=jnp.float32)
    q = qkv[:, :D]
    qk_ref[0] = q
    qk_ref[1] = qkv[:, D:2 * D]
    u_ref[...] = jnp.sum(q * we_ref[0:1, :], axis=1, keepdims=True)
    vlo_ref[...] = qkv[:, 2 * D:2 * D + DH]
    vhi_ref[...] = qkv[:, 2 * D + DH:]


def _out_body(np_ref, x_ref, we_ref, wo_ref, wgt_ref, o_ref):
    p0 = np_ref[0]
    p1 = np_ref[1]
    vsum = jnp.concatenate([p0[:, :DH], p1[:, :DH]], axis=1)
    denom = p0[:, DH:DH + 1]
    wea = p0[:, DH + 1:DH + 2]
    num = vsum + wea * we_ref[0:1, :]
    agg = num / (denom + 1e-9)
    msg = jnp.dot(agg, wo_ref[...], preferred_element_type=jnp.float32)
    xb = x_ref[...]
    wg1 = wgt_ref[0:1, :D]
    wg2 = wgt_ref[0:1, D:]
    gl = (jnp.sum(xb * wg1, axis=1, keepdims=True)
          + jnp.sum(msg * wg2, axis=1, keepdims=True))
    gate = jax.nn.sigmoid(gl)
    o_ref[...] = gate * xb + (1.0 - gate) * msg


CH_A = 16
CH_B = 8


def _chunk_vec(mch, blk, row, g):
    idx = lax.iota(jnp.int32, 16) + g * 16
    return plsc.load_gather(
        mch, [jnp.full((16,), blk, jnp.int32),
              jnp.full((16,), row, jnp.int32), idx])


def _sc_w_body(qk, gidx, u, meta, w_out,
               u_v, mch, gch, qkb0, qkb1, wch,
               sq0, sq1):
    c = lax.axis_index("c")
    s = lax.axis_index("s")
    wid = c * 16 + s
    start = wid * NB_A
    inv = jnp.float32(1.0 / math.sqrt(float(D)))

    pltpu.sync_copy(u, u_v)

    def fire(bi, qk_buf, sem_q):
        pltpu.async_copy(qk.at[gch.at[bi]], qk_buf, sem_q)

    def wait(bi, qk_buf, sem_q):
        pltpu.make_async_copy(qk.at[gch.at[bi]], qk_buf, sem_q).wait()

    lane = lax.iota(jnp.int32, 16)
    fifteen = jnp.full((16,), 15, jnp.int32)

    def compute(bi, qk_buf):
        for g in range(BA // 16):
            dst_g = _chunk_vec(mch, bi, 1, g)
            ea_g = plsc.bitcast(_chunk_vec(mch, bi, 2, g), jnp.float32)
            u_g = plsc.load_gather(u_v, [dst_g, jnp.zeros((16,), jnp.int32)])

            def edge_body(l, lg):
                eq = jnp.full((16,), g * 32, jnp.int32) + 2 * l
                p = [plsc.load_gather(qk_buf, [eq, lane + ch * 16])
                     * plsc.load_gather(qk_buf, [eq + 1, lane + ch * 16])
                     for ch in range(D // 16)]
                t = (((p[0] + p[1]) + (p[2] + p[3]))
                     + ((p[4] + p[5]) + (p[6] + p[7])))
                sp = jnp.take(plsc.cumsum(t), fifteen)
                return jnp.where(lane == l, sp, lg)

            lg = lax.fori_loop(0, 16, edge_body,
                               jnp.zeros((16,), jnp.float32), unroll=4)
            w = jnp.exp((lg + ea_g * u_g) * inv)
            wch[pl.ds(bi * BA + g * 16, 16)] = w

    def chunk_body(ch, carry):
        cb = start + ch * CH_A
        pltpu.sync_copy(meta.at[pl.ds(cb, CH_A)], mch)
        pltpu.sync_copy(gidx.at[pl.ds(cb, CH_A)], gch)
        fire(0, qkb0, sq0)

        def pair_body(i, carry2):
            fire(2 * i + 1, qkb1, sq1)
            wait(2 * i, qkb0, sq0)
            compute(2 * i, qkb0)

            @pl.when(i < CH_A // 2 - 1)
            def _():
                fire(2 * i + 2, qkb0, sq0)

            wait(2 * i + 1, qkb1, sq1)
            compute(2 * i + 1, qkb1)
            return carry2

        lax.fori_loop(0, CH_A // 2, pair_body, 0)
        pltpu.sync_copy(wch, w_out.at[pl.ds(cb * BA, CH_A * BA)])
        return carry

    lax.fori_loop(0, NB_A // CH_A, chunk_body, 0)


def _sc_scat_body(vlo, vhi, meta, w, zrows, numer_out,
                  mch, wch, vb0, vb1, sc0, sc1, di0, di1,
                  numer_sh, sv0, sv1, ss0, ss1):
    c = lax.axis_index("c")
    s = lax.axis_index("s")

    pltpu.sync_copy(zrows, numer_sh.at[pl.ds(s * RSUB, RSUB)])
    pltpu.sync_copy(zrows.at[pl.ds(0, B)], sc0)
    pltpu.sync_copy(zrows.at[pl.ds(0, B)], sc1)
    plsc.subcore_barrier()

    start = s * NB_B

    def fire(bi, vh_buf, sem_v):
        @pl.when(c == 0)
        def _():
            pltpu.async_copy(vlo.at[mch.at[bi, 0]], vh_buf, sem_v)

        @pl.when(c == 1)
        def _():
            pltpu.async_copy(vhi.at[mch.at[bi, 0]], vh_buf, sem_v)

    def wait(bi, vh_buf, sem_v):
        pltpu.make_async_copy(vlo.at[mch.at[bi, 0]], vh_buf, sem_v).wait()

    lane = lax.iota(jnp.int32, 16)
    zero16 = jnp.zeros((16,), jnp.float32)

    def compute(i, bi, vh_buf, scaled, dsti, sem_s):
        @pl.when(i > 1)
        def _():
            pltpu.make_async_copy(scaled, numer_sh.at[dsti], sem_s).wait()
        for g in range(B // 16):
            w_g = wch[pl.ds(bi * B + g * 16, 16)]
            ea_g = plsc.bitcast(_chunk_vec(mch, bi, 2, g), jnp.float32)
            dsti[pl.ds(g * 16, 16)] = _chunk_vec(mch, bi, 1, g)
            wea_g = w_g * ea_g

            def edge_body(l, carry3):
                e = jnp.full((16,), g * 16, jnp.int32) + l
                lsp = jnp.full((16,), 0, jnp.int32) + l
                spw = jnp.take(w_g, lsp)
                for ch in range(DH // 16):
                    vc = plsc.load_gather(vh_buf, [e, lane + ch * 16])
                    plsc.store_scatter(scaled, [e, lane + ch * 16], spw * vc)
                spwea = jnp.take(wea_g, lsp)
                tail = (jnp.where(lane == 0, spw, zero16)
                        + jnp.where(lane == 1, spwea, zero16))
                plsc.store_scatter(scaled, [e, lane + DH], tail)
                return carry3

            lax.fori_loop(0, 16, edge_body, 0, unroll=4)
        pltpu.async_copy(scaled, numer_sh.at[dsti], sem_s, add=True)

    def chunk_body(ch, carry):
        cb = start + ch * CH_B
        pltpu.sync_copy(meta.at[pl.ds(cb, CH_B)], mch)
        pltpu.sync_copy(w.at[pl.ds(cb * B, CH_B * B)], wch)
        fire(0, vb0, sv0)

        def pair_body(i, carry2):
            ii = ch * CH_B + 2 * i
            fire(2 * i + 1, vb1, sv1)
            wait(2 * i, vb0, sv0)
            compute(ii, 2 * i, vb0, sc0, di0, ss0)

            @pl.when(i < CH_B // 2 - 1)
            def _():
                fire(2 * i + 2, vb0, sv0)

            wait(2 * i + 1, vb1, sv1)
            compute(ii + 1, 2 * i + 1, vb1, sc1, di1, ss1)
            return carry2

        lax.fori_loop(0, CH_B // 2, pair_body, 0)
        return carry

    lax.fori_loop(0, NB_B // CH_B, chunk_body, 0)

    pltpu.make_async_copy(sc0, numer_sh.at[di0], ss0).wait()
    pltpu.make_async_copy(sc1, numer_sh.at[di1], ss1).wait()
    plsc.subcore_barrier()
    r0 = s * RSUB
    pltpu.sync_copy(numer_sh.at[pl.ds(r0, RSUB)],
                    numer_out.at[c, pl.ds(r0, RSUB)])


def kernel(x, edge_index, edge_attr, Wq, Wk, Wv, We, Wo, Wg):
    pad = EP - E
    srcp = jnp.concatenate([edge_index[0],
                            jnp.full((pad,), NP - 1, jnp.int32)])
    dstp = jnp.concatenate([edge_index[1],
                            jnp.full((pad,), NP - 1, jnp.int32)])
    eab = lax.bitcast_convert_type(
        jnp.concatenate([edge_attr[:, 0], jnp.zeros((pad,), jnp.float32)]),
        jnp.int32)
    meta = jnp.stack([srcp.reshape(NBLKP, B), dstp.reshape(NBLKP, B),
                      eab.reshape(NBLKP, B)], axis=1)
    meta_a = jnp.stack([srcp.reshape(NBLKA, BA), dstp.reshape(NBLKA, BA),
                        eab.reshape(NBLKA, BA)], axis=1)
    xp = jnp.concatenate([x, jnp.zeros((NP - N, D), jnp.float32)])
    wcat = jnp.concatenate([Wq, Wk, Wv], axis=1)
    zrows = jnp.zeros((RSUB, DP), jnp.float32)

    qk3, u, vlo, vhi = pl.pallas_call(
        _proj_body,
        grid=(NP // R_TC,),
        in_specs=[
            pl.BlockSpec((R_TC, D), lambda i: (i, 0)),
            pl.BlockSpec((D, 3 * D), lambda i: (0, 0)),
            pl.BlockSpec((1, D), lambda i: (0, 0)),
        ],
        out_specs=[
            pl.BlockSpec((2, R_TC, D), lambda i: (0, i, 0)),
            pl.BlockSpec((R_TC, 1), lambda i: (i, 0)),
            pl.BlockSpec((R_TC, DH), lambda i: (i, 0)),
            pl.BlockSpec((R_TC, DH), lambda i: (i, 0)),
        ],
        out_shape=[
            jax.ShapeDtypeStruct((2, NP, D), jnp.float32),
            jax.ShapeDtypeStruct((NP, 1), jnp.float32),
            jax.ShapeDtypeStruct((NP, DH), jnp.float32),
            jax.ShapeDtypeStruct((NP, DH), jnp.float32),
        ],
    )(xp, wcat, We)
    qk2 = qk3.reshape(2 * NP, D)
    gidx = jnp.stack([dstp.reshape(NBLKA, BA),
                      srcp.reshape(NBLKA, BA) + NP],
                     axis=2).reshape(NBLKA, 2 * BA)

    mesh = plsc.VectorSubcoreMesh(core_axis_name="c", subcore_axis_name="s",
                                  num_cores=2, num_subcores=16)
    sc_params = pltpu.CompilerParams(use_tc_tiling_on_sc=False,
                                     needs_layout_passes=False)

    w = pl.kernel(
        _sc_w_body,
        out_type=jax.ShapeDtypeStruct((EP,), jnp.float32),
        mesh=mesh,
        compiler_params=sc_params,
        scratch_types=[
            pltpu.VMEM((NP, 1), jnp.float32),
            pltpu.VMEM((CH_A, 3, BA), jnp.int32),
            pltpu.VMEM((CH_A, 2 * BA), jnp.int32),
            pltpu.VMEM((2 * BA, D), jnp.float32),
            pltpu.VMEM((2 * BA, D), jnp.float32),
            pltpu.VMEM((CH_A * BA,), jnp.float32),
            pltpu.SemaphoreType.DMA,
            pltpu.SemaphoreType.DMA,
        ],
    )(qk2, gidx, u, meta_a)

    numer = pl.kernel(
        _sc_scat_body,
        out_type=jax.ShapeDtypeStruct((2, NP, DP), jnp.float32),
        mesh=plsc.VectorSubcoreMesh(core_axis_name="c", subcore_axis_name="s",
                                    num_cores=2, num_subcores=16),
        compiler_params=sc_params,
        scratch_types=[
            pltpu.VMEM((CH_B, 3, B), jnp.int32),
            pltpu.VMEM((CH_B * B,), jnp.float32),
            pltpu.VMEM((B, DH), jnp.float32),
            pltpu.VMEM((B, DH), jnp.float32),
            pltpu.VMEM((B, DP), jnp.float32),
            pltpu.VMEM((B, DP), jnp.float32),
            pltpu.VMEM((B,), jnp.int32),
            pltpu.VMEM((B,), jnp.int32),
            pltpu.VMEM_SHARED((NP, DP), jnp.float32),
            pltpu.SemaphoreType.DMA,
            pltpu.SemaphoreType.DMA,
            pltpu.SemaphoreType.DMA,
            pltpu.SemaphoreType.DMA,
        ],
    )(vlo, vhi, meta, w, zrows)

    out = pl.pallas_call(
        _out_body,
        grid=(N // R_TC2,),
        in_specs=[
            pl.BlockSpec((2, R_TC2, DP), lambda i: (0, i, 0)),
            pl.BlockSpec((R_TC2, D), lambda i: (i, 0)),
            pl.BlockSpec((1, D), lambda i: (0, 0)),
            pl.BlockSpec((D, D), lambda i: (0, 0)),
            pl.BlockSpec((1, 2 * D), lambda i: (0, 0)),
        ],
        out_specs=pl.BlockSpec((R_TC2, D), lambda i: (i, 0)),
        out_shape=jax.ShapeDtypeStruct((N, D), jnp.float32),
    )(numer, x, We, Wo, Wg.T)

    return out


# speedup vs baseline: 1.2159x; 1.2159x over previous
import math

import jax
import jax.numpy as jnp
from jax import lax
from jax.experimental import pallas as pl
from jax.experimental.pallas import tpu as pltpu
from jax.experimental.pallas import tpu_sc as plsc

N = 10000
NP = 10240
E = 320000
D = 128
DH = D // 2
DP = 80
B = 128
NBLKP = 2560
EP = NBLKP * B
RSUB = NP // 16
R_TC = 2048
R_TC2 = 2000
BA = 64
NBLKA = EP // BA
NB_A = NBLKA // 32
NB_B = NBLKP // 16


def _proj_body(x_ref, wcat_ref, we_ref, q_ref, k_ref, u_ref, vlo_ref, vhi_ref):
    xb = x_ref[...]
    qkv = jnp.dot(xb, wcat_ref[...], preferred_element_type=jnp.float32)
    q = qkv[:, :D]
    q_ref[...] = q
    k_ref[...] = qkv[:, D:2 * D]
    u_ref[...] = jnp.sum(q * we_ref[0:1, :], axis=1, keepdims=True)
    vlo_ref[...] = qkv[:, 2 * D:2 * D + DH]
    vhi_ref[...] = qkv[:, 2 * D + DH:]


def _out_body(np_ref, x_ref, we_ref, wo_ref, wgt_ref, o_ref):
    p0 = np_ref[0]
    p1 = np_ref[1]
    vsum = jnp.concatenate([p0[:, :DH], p1[:, :DH]], axis=1)
    denom = p0[:, DH:DH + 1]
    wea = p0[:, DH + 1:DH + 2]
    num = vsum + wea * we_ref[0:1, :]
    agg = num / (denom + 1e-9)
    msg = jnp.dot(agg, wo_ref[...], preferred_element_type=jnp.float32)
    xb = x_ref[...]
    wg1 = wgt_ref[0:1, :D]
    wg2 = wgt_ref[0:1, D:]
    gl = (jnp.sum(xb * wg1, axis=1, keepdims=True)
          + jnp.sum(msg * wg2, axis=1, keepdims=True))
    gate = jax.nn.sigmoid(gl)
    o_ref[...] = gate * xb + (1.0 - gate) * msg


CH_A = 16
CH_B = 16


def _chunk_vec(mch, blk, row, g):
    idx = lax.iota(jnp.int32, 16) + g * 16
    return plsc.load_gather(
        mch, [jnp.full((16,), blk, jnp.int32),
              jnp.full((16,), row, jnp.int32), idx])


def _sc_w_body(q, k, u, meta, w_out,
               u_v, mch, qb0, qb1, kb0, kb1, wch,
               sq0, sq1, sk0, sk1):
    c = lax.axis_index("c")
    s = lax.axis_index("s")
    wid = c * 16 + s
    start = wid * NB_A
    inv = jnp.float32(1.0 / math.sqrt(float(D)))

    pltpu.sync_copy(u, u_v)

    def fire(bi, q_buf, k_buf, sem_q, sem_k):
        pltpu.async_copy(q.at[mch.at[bi, 1]], q_buf, sem_q)
        pltpu.async_copy(k.at[mch.at[bi, 0]], k_buf, sem_k)

    def wait(bi, q_buf, k_buf, sem_q, sem_k):
        pltpu.make_async_copy(q.at[mch.at[bi, 1]], q_buf, sem_q).wait()
        pltpu.make_async_copy(k.at[mch.at[bi, 0]], k_buf, sem_k).wait()

    lane = lax.iota(jnp.int32, 16)
    fifteen = jnp.full((16,), 15, jnp.int32)

    def compute(bi, q_buf, k_buf):
        for g in range(BA // 16):
            dst_g = _chunk_vec(mch, bi, 1, g)
            ea_g = plsc.bitcast(_chunk_vec(mch, bi, 2, g), jnp.float32)
            u_g = plsc.load_gather(u_v, [dst_g, jnp.zeros((16,), jnp.int32)])

            def edge_body(l, lg):
                e = jnp.full((16,), g * 16, jnp.int32) + l
                p = [plsc.load_gather(q_buf, [e, lane + ch * 16])
                     * plsc.load_gather(k_buf, [e, lane + ch * 16])
                     for ch in range(D // 16)]
                t = (((p[0] + p[1]) + (p[2] + p[3]))
                     + ((p[4] + p[5]) + (p[6] + p[7])))
                sp = jnp.take(plsc.cumsum(t), fifteen)
                return jnp.where(lane == l, sp, lg)

            lg = lax.fori_loop(0, 16, edge_body,
                               jnp.zeros((16,), jnp.float32), unroll=2)
            w = jnp.exp((lg + ea_g * u_g) * inv)
            wch[pl.ds(bi * BA + g * 16, 16)] = w

    def chunk_body(ch, carry):
        cb = start + ch * CH_A
        pltpu.sync_copy(meta.at[pl.ds(cb, CH_A)], mch)
        fire(0, qb0, kb0, sq0, sk0)

        def pair_body(i, carry2):
            fire(2 * i + 1, qb1, kb1, sq1, sk1)
            wait(2 * i, qb0, kb0, sq0, sk0)
            compute(2 * i, qb0, kb0)

            @pl.when(i < CH_A // 2 - 1)
            def _():
                fire(2 * i + 2, qb0, kb0, sq0, sk0)

            wait(2 * i + 1, qb1, kb1, sq1, sk1)
            compute(2 * i + 1, qb1, kb1)
            return carry2

        lax.fori_loop(0, CH_A // 2, pair_body, 0)
        pltpu.sync_copy(wch, w_out.at[pl.ds(cb * BA, CH_A * BA)])
        return carry

    lax.fori_loop(0, NB_A // CH_A, chunk_body, 0)


def _sc_scat_body(vlo, vhi, meta, w, zrows, numer_out,
                  mch, wch, vb0, vb1, vb2, vb3, sc0, sc1, di0, di1,
                  numer_sh, sv0, sv1, sv2, sv3, ss0, ss1):
    c = lax.axis_index("c")
    s = lax.axis_index("s")

    pltpu.sync_copy(zrows, numer_sh.at[pl.ds(s * RSUB, RSUB)])
    pltpu.sync_copy(zrows.at[pl.ds(0, B)], sc0)
    pltpu.sync_copy(zrows.at[pl.ds(0, B)], sc1)
    plsc.subcore_barrier()

    start = s * NB_B

    def fire(bi, vh_buf, sem_v):
        @pl.when(c == 0)
        def _():
            pltpu.async_copy(vlo.at[mch.at[bi, 0]], vh_buf, sem_v)

        @pl.when(c == 1)
        def _():
            pltpu.async_copy(vhi.at[mch.at[bi, 0]], vh_buf, sem_v)

    def wait(bi, vh_buf, sem_v):
        pltpu.make_async_copy(vlo.at[mch.at[bi, 0]], vh_buf, sem_v).wait()

    lane = lax.iota(jnp.int32, 16)
    zero16 = jnp.zeros((16,), jnp.float32)

    def compute(i, bi, vh_buf, scaled, dsti, sem_s):
        @pl.when(i > 1)
        def _():
            pltpu.make_async_copy(scaled, numer_sh.at[dsti], sem_s).wait()
        for g in range(B // 16):
            w_g = wch[pl.ds(bi * B + g * 16, 16)]
            ea_g = plsc.bitcast(_chunk_vec(mch, bi, 2, g), jnp.float32)
            dsti[pl.ds(g * 16, 16)] = _chunk_vec(mch, bi, 1, g)
            wea_g = w_g * ea_g

            def edge_body(l, carry3):
                e = jnp.full((16,), g * 16, jnp.int32) + l
                lsp = jnp.full((16,), 0, jnp.int32) + l
                spw = jnp.take(w_g, lsp)
                for ch in range(DH // 16):
                    vc = plsc.load_gather(vh_buf, [e, lane + ch * 16])
                    plsc.store_scatter(scaled, [e, lane + ch * 16], spw * vc)
                spwea = jnp.take(wea_g, lsp)
                tail = (jnp.where(lane == 0, spw, zero16)
                        + jnp.where(lane == 1, spwea, zero16))
                plsc.store_scatter(scaled, [e, lane + DH], tail)
                return carry3

            lax.fori_loop(0, 16, edge_body, 0, unroll=2)
        pltpu.async_copy(scaled, numer_sh.at[dsti], sem_s, add=True)

    def chunk_body(ch, carry):
        cb = start + ch * CH_B
        pltpu.sync_copy(meta.at[pl.ds(cb, CH_B)], mch)
        pltpu.sync_copy(w.at[pl.ds(cb * B, CH_B * B)], wch)
        fire(0, vb0, sv0)
        fire(1, vb1, sv1)
        fire(2, vb2, sv2)

        def quad_body(i, carry2):
            b = 4 * i
            ii = ch * CH_B + b
            last = i >= CH_B // 4 - 1
            fire(b + 3, vb3, sv3)
            wait(b, vb0, sv0)
            compute(ii, b, vb0, sc0, di0, ss0)

            @pl.when(jnp.logical_not(last))
            def _():
                fire(b + 4, vb0, sv0)

            wait(b + 1, vb1, sv1)
            compute(ii + 1, b + 1, vb1, sc1, di1, ss1)

            @pl.when(jnp.logical_not(last))
            def _():
                fire(b + 5, vb1, sv1)

            wait(b + 2, vb2, sv2)
            compute(ii + 2, b + 2, vb2, sc0, di0, ss0)

            @pl.when(jnp.logical_not(last))
            def _():
                fire(b + 6, vb2, sv2)

            wait(b + 3, vb3, sv3)
            compute(ii + 3, b + 3, vb3, sc1, di1, ss1)
            return carry2

        lax.fori_loop(0, CH_B // 4, quad_body, 0)
        return carry

    lax.fori_loop(0, NB_B // CH_B, chunk_body, 0)

    pltpu.make_async_copy(sc0, numer_sh.at[di0], ss0).wait()
    pltpu.make_async_copy(sc1, numer_sh.at[di1], ss1).wait()
    plsc.subcore_barrier()
    r0 = s * RSUB
    pltpu.sync_copy(numer_sh.at[pl.ds(r0, RSUB)],
                    numer_out.at[c, pl.ds(r0, RSUB)])


def kernel(x, edge_index, edge_attr, Wq, Wk, Wv, We, Wo, Wg):
    pad = EP - E
    srcp = jnp.concatenate([edge_index[0],
                            jnp.full((pad,), NP - 1, jnp.int32)])
    dstp = jnp.concatenate([edge_index[1],
                            jnp.full((pad,), NP - 1, jnp.int32)])
    eab = lax.bitcast_convert_type(
        jnp.concatenate([edge_attr[:, 0], jnp.zeros((pad,), jnp.float32)]),
        jnp.int32)
    meta = jnp.stack([srcp.reshape(NBLKP, B), dstp.reshape(NBLKP, B),
                      eab.reshape(NBLKP, B)], axis=1)
    meta_a = jnp.stack([srcp.reshape(NBLKA, BA), dstp.reshape(NBLKA, BA),
                        eab.reshape(NBLKA, BA)], axis=1)
    xp = jnp.concatenate([x, jnp.zeros((NP - N, D), jnp.float32)])
    wcat = jnp.concatenate([Wq, Wk, Wv], axis=1)
    zrows = jnp.zeros((RSUB, DP), jnp.float32)

    q, k, u, vlo, vhi = pl.pallas_call(
        _proj_body,
        grid=(NP // R_TC,),
        in_specs=[
            pl.BlockSpec((R_TC, D), lambda i: (i, 0)),
            pl.BlockSpec((D, 3 * D), lambda i: (0, 0)),
            pl.BlockSpec((1, D), lambda i: (0, 0)),
        ],
        out_specs=[
            pl.BlockSpec((R_TC, D), lambda i: (i, 0)),
            pl.BlockSpec((R_TC, D), lambda i: (i, 0)),
            pl.BlockSpec((R_TC, 1), lambda i: (i, 0)),
            pl.BlockSpec((R_TC, DH), lambda i: (i, 0)),
            pl.BlockSpec((R_TC, DH), lambda i: (i, 0)),
        ],
        out_shape=[
            jax.ShapeDtypeStruct((NP, D), jnp.float32),
            jax.ShapeDtypeStruct((NP, D), jnp.float32),
            jax.ShapeDtypeStruct((NP, 1), jnp.float32),
            jax.ShapeDtypeStruct((NP, DH), jnp.float32),
            jax.ShapeDtypeStruct((NP, DH), jnp.float32),
        ],
    )(xp, wcat, We)

    mesh = plsc.VectorSubcoreMesh(core_axis_name="c", subcore_axis_name="s",
                                  num_cores=2, num_subcores=16)
    sc_params = pltpu.CompilerParams(use_tc_tiling_on_sc=False,
                                     needs_layout_passes=False)

    w = pl.kernel(
        _sc_w_body,
        out_type=jax.ShapeDtypeStruct((EP,), jnp.float32),
        mesh=mesh,
        compiler_params=sc_params,
        scratch_types=[
            pltpu.VMEM((NP, 1), jnp.float32),
            pltpu.VMEM((CH_A, 3, BA), jnp.int32),
            pltpu.VMEM((BA, D), jnp.float32),
            pltpu.VMEM((BA, D), jnp.float32),
            pltpu.VMEM((BA, D), jnp.float32),
            pltpu.VMEM((BA, D), jnp.float32),
            pltpu.VMEM((CH_A * BA,), jnp.float32),
            pltpu.SemaphoreType.DMA,
            pltpu.SemaphoreType.DMA,
            pltpu.SemaphoreType.DMA,
            pltpu.SemaphoreType.DMA,
        ],
    )(q, k, u, meta_a)

    numer = pl.kernel(
        _sc_scat_body,
        out_type=jax.ShapeDtypeStruct((2, NP, DP), jnp.float32),
        mesh=plsc.VectorSubcoreMesh(core_axis_name="c", subcore_axis_name="s",
                                    num_cores=2, num_subcores=16),
        compiler_params=sc_params,
        scratch_types=[
            pltpu.VMEM((CH_B, 3, B), jnp.int32),
            pltpu.VMEM((CH_B * B,), jnp.float32),
            pltpu.VMEM((B, DH), jnp.float32),
            pltpu.VMEM((B, DH), jnp.float32),
            pltpu.VMEM((B, DH), jnp.float32),
            pltpu.VMEM((B, DH), jnp.float32),
            pltpu.VMEM((B, DP), jnp.float32),
            pltpu.VMEM((B, DP), jnp.float32),
            pltpu.VMEM((B,), jnp.int32),
            pltpu.VMEM((B,), jnp.int32),
            pltpu.VMEM_SHARED((NP, DP), jnp.float32),
            pltpu.SemaphoreType.DMA,
            pltpu.SemaphoreType.DMA,
            pltpu.SemaphoreType.DMA,
            pltpu.SemaphoreType.DMA,
            pltpu.SemaphoreType.DMA,
            pltpu.SemaphoreType.DMA,
        ],
    )(vlo, vhi, meta, w, zrows)

    out = pl.pallas_call(
        _out_body,
        grid=(N // R_TC2,),
        in_specs=[
            pl.BlockSpec((2, R_TC2, DP), lambda i: (0, i, 0)),
            pl.BlockSpec((R_TC2, D), lambda i: (i, 0)),
            pl.BlockSpec((1, D), lambda i: (0, 0)),
            pl.BlockSpec((D, D), lambda i: (0, 0)),
            pl.BlockSpec((1, 2 * D), lambda i: (0, 0)),
        ],
        out_specs=pl.BlockSpec((R_TC2, D), lambda i: (i, 0)),
        out_shape=jax.ShapeDtypeStruct((N, D), jnp.float32),
    )(numer, x, We, Wo, Wg.T)

    return out
